# Initial kernel scaffold; baseline (speedup 1.0000x reference)
import jax
import jax.numpy as jnp
from jax.experimental import pallas as pl


def kernel(x, edge_index, W1l, b1l, W1r, W2l, b2l, W2r, Wfc1, bfc1, Wfc2, bfc2):
    raise NotImplementedError("write your pallas kernel here")


# speedup vs baseline: 9.7149x; 9.7149x over previous
import functools

import jax
import jax.numpy as jnp
from jax import lax
from jax.experimental import pallas as pl
from jax.experimental.pallas import tpu as pltpu
from jax.experimental.pallas import tpu_sc as plsc

N = 10000
E = 320000
D = 128
H = 256
LH = 256

NC = 2
NS = 16
NW = NC * NS
GROUP = 80
ROWS = E // GROUP
RPT = ROWS // NW
NPT = 624
NTAIL = N - NS * NPT


def _sc_aggregate(x, src2d, dst2d, zeros_nd, zeros_n, ones_g):
    mesh = plsc.VectorSubcoreMesh(core_axis_name="c", subcore_axis_name="s")

    @functools.partial(
        pl.kernel,
        mesh=mesh,
        out_type=[
            jax.ShapeDtypeStruct((NC, N, D), jnp.float32),
            jax.ShapeDtypeStruct((NC, N), jnp.float32),
        ],
        scratch_types=[
            pltpu.VMEM((RPT, GROUP), jnp.int32),
            pltpu.VMEM((RPT, GROUP), jnp.int32),
            pltpu.VMEM((GROUP, D), jnp.float32),
            pltpu.VMEM((GROUP,), jnp.float32),
            pltpu.VMEM_SHARED((N, D), jnp.float32),
            pltpu.VMEM_SHARED((N,), jnp.float32),
            pltpu.SemaphoreType.DMA,
        ],
    )
    def kern(x_hbm, src_hbm, dst_hbm, z2_hbm, z1_hbm, ones_hbm,
             agg_hbm, deg_hbm,
             idx_s, idx_d, rows_v, ones_v, agg_sh, deg_sh, sem):
        c = lax.axis_index("c")
        s = lax.axis_index("s")
        w = c * NS + s
        nbase = pl.multiple_of(s * NPT, 8)
        pltpu.sync_copy(z2_hbm.at[pl.ds(nbase, NPT)],
                        agg_sh.at[pl.ds(nbase, NPT)])

        @pl.when(s == 0)
        def _():
            pltpu.sync_copy(z2_hbm.at[pl.ds(NS * NPT, NTAIL)],
                            agg_sh.at[pl.ds(NS * NPT, NTAIL)])
            pltpu.sync_copy(z1_hbm, deg_sh)

        pltpu.sync_copy(ones_hbm, ones_v)
        pltpu.sync_copy(src_hbm.at[w], idx_s)
        pltpu.sync_copy(dst_hbm.at[w], idx_d)
        plsc.subcore_barrier()

        def edge_step(j, carry):
            pltpu.async_copy(x_hbm.at[idx_s.at[j]], rows_v, sem).wait()
            pltpu.sync_copy(rows_v, agg_sh.at[idx_d.at[j]], add=True)
            pltpu.sync_copy(ones_v, deg_sh.at[idx_d.at[j]], add=True)
            return carry

        lax.fori_loop(0, RPT, edge_step, 0)
        plsc.subcore_barrier()

        pltpu.sync_copy(agg_sh.at[pl.ds(nbase, NPT)],
                        agg_hbm.at[c, pl.ds(nbase, NPT)])

        @pl.when(s == 0)
        def _():
            pltpu.sync_copy(agg_sh.at[pl.ds(NS * NPT, NTAIL)],
                            agg_hbm.at[c, pl.ds(NS * NPT, NTAIL)])
            pltpu.sync_copy(deg_sh, deg_hbm.at[c])

    return kern(x, src2d, dst2d, zeros_nd, zeros_n, ones_g)


def _sc_edge_scalar(svec, src2d, dst2d, zeros_n):
    mesh = plsc.VectorSubcoreMesh(core_axis_name="c", subcore_axis_name="s")

    @functools.partial(
        pl.kernel,
        mesh=mesh,
        out_type=jax.ShapeDtypeStruct((NC, N), jnp.float32),
        scratch_types=[
            pltpu.VMEM((RPT, GROUP), jnp.int32),
            pltpu.VMEM((RPT, GROUP), jnp.int32),
            pltpu.VMEM((GROUP,), jnp.float32),
            pltpu.VMEM_SHARED((N,), jnp.float32),
            pltpu.SemaphoreType.DMA,
        ],
    )
    def kern(s_hbm, src_hbm, dst_hbm, z1_hbm, t_hbm,
             idx_s, idx_d, vals_v, t_sh, sem):
        c = lax.axis_index("c")
        s = lax.axis_index("s")
        w = c * NS + s

        @pl.when(s == 0)
        def _():
            pltpu.sync_copy(z1_hbm, t_sh)

        pltpu.sync_copy(src_hbm.at[w], idx_s)
        pltpu.sync_copy(dst_hbm.at[w], idx_d)
        plsc.subcore_barrier()

        def edge_step(j, carry):
            pltpu.async_copy(s_hbm.at[idx_s.at[j]], vals_v, sem).wait()
            pltpu.sync_copy(vals_v, t_sh.at[idx_d.at[j]], add=True)
            return carry

        lax.fori_loop(0, RPT, edge_step, 0)
        plsc.subcore_barrier()

        @pl.when(s == 0)
        def _():
            pltpu.sync_copy(t_sh, t_hbm.at[c])

    return kern(svec, src2d, dst2d, zeros_n)


BN = 1000


def _tc_dense1(agg0, agg1, deg0, deg1, x, w1lt, b1l_r, w1rt, w2s):
    def body(agg0_r, agg1_r, deg0_r, deg1_r, x_r, w1l_r, b1l_r_, w1r_r,
             w2s_r, sh_r, degc_r):
        degc = jnp.maximum(deg0_r[...] + deg1_r[...], 1.0)
        mean = (agg0_r[...] + agg1_r[...]) / degc
        h = (jnp.dot(mean, w1l_r[...], preferred_element_type=jnp.float32)
             + b1l_r_[...]
             + jnp.dot(x_r[...], w1r_r[...], preferred_element_type=jnp.float32))
        h = jnp.maximum(h, 0.0)
        sh_r[...] = jnp.dot(h, w2s_r[...], preferred_element_type=jnp.float32)
        degc_r[...] = degc

    grid = (N // BN,)
    return pl.pallas_call(
        body,
        grid=grid,
        in_specs=[
            pl.BlockSpec((BN, D), lambda i: (i, 0)),
            pl.BlockSpec((BN, D), lambda i: (i, 0)),
            pl.BlockSpec((BN, 1), lambda i: (i, 0)),
            pl.BlockSpec((BN, 1), lambda i: (i, 0)),
            pl.BlockSpec((BN, D), lambda i: (i, 0)),
            pl.BlockSpec((D, H), lambda i: (0, 0)),
            pl.BlockSpec((1, H), lambda i: (0, 0)),
            pl.BlockSpec((D, H), lambda i: (0, 0)),
            pl.BlockSpec((H, 2), lambda i: (0, 0)),
        ],
        out_specs=[
            pl.BlockSpec((BN, 2), lambda i: (i, 0)),
            pl.BlockSpec((BN, 1), lambda i: (i, 0)),
        ],
        out_shape=[
            jax.ShapeDtypeStruct((N, 2), jnp.float32),
            jax.ShapeDtypeStruct((N, 1), jnp.float32),
        ],
    )(agg0, agg1, deg0, deg1, x, w1lt, b1l_r, w1rt, w2s)


def _tc_dense2(t0, t1, degc, hr, b2, wfc1, bfc1, wfc2, bfc2):
    def body(t0_r, t1_r, degc_r, hr_r, b2_r, wfc1_r, bfc1_r, wfc2_r, bfc2_r,
             p_r):
        v = jnp.maximum((t0_r[...] + t1_r[...]) / degc_r[...]
                        + b2_r[0, 0] + hr_r[...], 0.0)
        z = (jnp.dot(wfc1_r[...], v, preferred_element_type=jnp.float32)
             + bfc1_r[...])
        p_r[...] = (jnp.dot(wfc2_r[...], z,
                            preferred_element_type=jnp.float32)
                    + bfc2_r[...])

    return pl.pallas_call(
        body,
        out_shape=jax.ShapeDtypeStruct((1, 1), jnp.float32),
    )(t0, t1, degc, hr, b2, wfc1, bfc1, wfc2, bfc2)


def kernel(x, edge_index, W1l, b1l, W1r, W2l, b2l, W2r, Wfc1, bfc1, Wfc2, bfc2):
    src2d = edge_index[0].reshape(NW, RPT, GROUP)
    dst2d = edge_index[1].reshape(NW, RPT, GROUP)
    zeros_nd = jnp.zeros((N, D), jnp.float32)
    zeros_n = jnp.zeros((N,), jnp.float32)
    ones_g = jnp.ones((GROUP,), jnp.float32)

    agg2, deg2 = _sc_aggregate(x, src2d, dst2d, zeros_nd, zeros_n, ones_g)

    sh, degc = _tc_dense1(
        agg2[0], agg2[1],
        deg2[0].reshape(N, 1), deg2[1].reshape(N, 1),
        x, W1l.T, b1l.reshape(1, H), W1r.T,
        jnp.concatenate([W2l, W2r], axis=0).T,
    )

    s_vec = sh[:, 0]
    hr = sh[:, 1:2]

    t2 = _sc_edge_scalar(s_vec, src2d, dst2d, zeros_n)

    p = _tc_dense2(
        t2[0].reshape(N, 1), t2[1].reshape(N, 1), degc, hr,
        b2l.reshape(1, 1), Wfc1, bfc1.reshape(LH, 1), Wfc2,
        bfc2.reshape(1, 1),
    )
    return jnp.reshape(p, ())


# speedup vs baseline: 18.9080x; 1.9463x over previous
import functools

import jax
import jax.numpy as jnp
from jax import lax
from jax.experimental import pallas as pl
from jax.experimental.pallas import tpu as pltpu
from jax.experimental.pallas import tpu_sc as plsc

N = 10000
E = 320000
D = 128
H = 256
LH = 256

NC = 2
NS = 16
NW = NC * NS
GROUP = 80
ROWS = E // GROUP
RPT = ROWS // NW
NPT = 624
NTAIL = N - NS * NPT
NBUF = 5
GLOOP = RPT // NBUF
GA = 40
ROWSA = E // GA
RPTA = ROWSA // NW
CH = 25
NCHUNK = RPTA // CH


def _sc_aggregate(x, srcA, dstA, zeros_nd, zeros_n, ones_g):
    mesh = plsc.VectorSubcoreMesh(core_axis_name="c", subcore_axis_name="s")

    @functools.partial(
        pl.kernel,
        mesh=mesh,
        out_type=[
            jax.ShapeDtypeStruct((NC, N, D), jnp.float32),
            jax.ShapeDtypeStruct((NC, N), jnp.float32),
        ],
        scratch_types=[
            [pltpu.VMEM((CH, GA), jnp.int32)] * 2,
            [pltpu.VMEM((CH, GA), jnp.int32)] * 2,
            [pltpu.VMEM((GA, D), jnp.float32)] * NBUF,
            pltpu.VMEM((GA,), jnp.float32),
            pltpu.VMEM_SHARED((N, D), jnp.float32),
            pltpu.VMEM_SHARED((N,), jnp.float32),
            [pltpu.SemaphoreType.DMA] * NBUF,
            [pltpu.SemaphoreType.DMA] * NBUF,
            [pltpu.SemaphoreType.DMA] * 2,
        ],
    )
    def kern(x_hbm, src_hbm, dst_hbm, z2_hbm, z1_hbm, ones_hbm,
             agg_hbm, deg_hbm,
             idx_s2, idx_d2, rows, ones_v, agg_sh, deg_sh, gsem, ssem, isem):
        c = lax.axis_index("c")
        s = lax.axis_index("s")
        w = c * NS + s
        nbase = pl.multiple_of(s * NPT, 8)
        pltpu.sync_copy(z2_hbm.at[pl.ds(nbase, NPT)],
                        agg_sh.at[pl.ds(nbase, NPT)])

        @pl.when(s == 0)
        def _():
            pltpu.sync_copy(z2_hbm.at[pl.ds(NS * NPT, NTAIL)],
                            agg_sh.at[pl.ds(NS * NPT, NTAIL)])
            pltpu.sync_copy(z1_hbm, deg_sh)

        pltpu.sync_copy(ones_hbm, ones_v)

        def load_chunk(u, p):
            pltpu.async_copy(src_hbm.at[w, u], idx_s2[p], isem[p])
            pltpu.async_copy(dst_hbm.at[w, u], idx_d2[p], isem[p])

        def wait_chunk(u, p):
            pltpu.make_async_copy(src_hbm.at[w, u], idx_s2[p],
                                  isem[p]).wait()
            pltpu.make_async_copy(dst_hbm.at[w, u], idx_d2[p],
                                  isem[p]).wait()

        load_chunk(0, 0)
        wait_chunk(0, 0)
        load_chunk(1, 1)
        plsc.subcore_barrier()

        def start_gather(ix, r, b):
            pltpu.async_copy(x_hbm.at[ix.at[r]], rows[b], gsem[b])

        def wait_gather(ix, r, b):
            pltpu.make_async_copy(x_hbm.at[ix.at[r]], rows[b],
                                  gsem[b]).wait()

        def start_scatter(ix, r, b):
            pltpu.async_copy(rows[b], agg_sh.at[ix.at[r]], ssem[b],
                             add=True)
            pltpu.async_copy(ones_v, deg_sh.at[ix.at[r]], ssem[b],
                             add=True)

        def wait_scatter(ix, r, b):
            pltpu.make_async_copy(rows[b], agg_sh.at[ix.at[r]],
                                  ssem[b]).wait()
            pltpu.make_async_copy(ones_v, deg_sh.at[ix.at[r]],
                                  ssem[b]).wait()

        for b in range(3):
            start_gather(idx_s2[0], b, b)

        def pair_body(q, carry):
            for p in range(2):
                u = q * 2 + p
                isC, idC = idx_s2[p], idx_d2[p]
                isN, idN = idx_s2[1 - p], idx_d2[1 - p]
                for loc in (0, 1):
                    wait_gather(isC, loc, loc)
                    start_scatter(idC, loc, loc)
                    if p == 1:
                        wait_scatter(idN, 23 + loc, 3 + loc)
                    else:
                        @pl.when(q > 0)
                        def _():
                            wait_scatter(idN, 23 + loc, 3 + loc)
                    start_gather(isC, loc + 3, loc + 3)
                if p == 1:
                    @pl.when(q < (NCHUNK // 2) - 1)
                    def _():
                        load_chunk(u + 1, 0)
                else:
                    @pl.when(q > 0)
                    def _():
                        load_chunk(u + 1, 1)
                def mid(g, carry2):
                    for b in range(NBUF):
                        loc = 2 + g * NBUF + b
                        rb = (2 + b) % NBUF
                        wait_gather(isC, loc, rb)
                        start_scatter(idC, loc, rb)
                        wait_scatter(idC, loc - 2, b)
                        start_gather(isC, loc + 3, b)
                    return carry2

                lax.fori_loop(0, 4, mid, 0)
                last = (p == 1)
                for k in range(3):
                    loc = 22 + k
                    if k == 0:
                        if last:
                            @pl.when(q < (NCHUNK // 2) - 1)
                            def _():
                                wait_chunk(u + 1, 1 - p)
                        else:
                            wait_chunk(u + 1, 1 - p)
                    wait_gather(isC, loc, loc - 20)
                    start_scatter(idC, loc, loc - 20)
                    wait_scatter(idC, loc - 2, k)
                    if last:
                        @pl.when(q < (NCHUNK // 2) - 1)
                        def _():
                            start_gather(isN, k, k)
                    else:
                        start_gather(isN, k, k)
            return carry

        lax.fori_loop(0, NCHUNK // 2, pair_body, 0)
        wait_scatter(idx_d2[1], 23, 3)
        wait_scatter(idx_d2[1], 24, 4)
        plsc.subcore_barrier()

        pltpu.sync_copy(agg_sh.at[pl.ds(nbase, NPT)],
                        agg_hbm.at[c, pl.ds(nbase, NPT)])

        @pl.when(s == 0)
        def _():
            pltpu.sync_copy(agg_sh.at[pl.ds(NS * NPT, NTAIL)],
                            agg_hbm.at[c, pl.ds(NS * NPT, NTAIL)])
            pltpu.sync_copy(deg_sh, deg_hbm.at[c])

    return kern(x, srcA, dstA, zeros_nd, zeros_n, ones_g)


def _sc_edge_scalar(svec, src2d, dst2d, zeros_n):
    mesh = plsc.VectorSubcoreMesh(core_axis_name="c", subcore_axis_name="s")

    @functools.partial(
        pl.kernel,
        mesh=mesh,
        out_type=jax.ShapeDtypeStruct((NC, N), jnp.float32),
        scratch_types=[
            pltpu.VMEM((RPT, GROUP), jnp.int32),
            pltpu.VMEM((RPT, GROUP), jnp.int32),
            [pltpu.VMEM((GROUP,), jnp.float32)] * NBUF,
            pltpu.VMEM_SHARED((N,), jnp.float32),
            pltpu.VMEM_SHARED((N,), jnp.float32),
            [pltpu.SemaphoreType.DMA] * NBUF,
            [pltpu.SemaphoreType.DMA] * NBUF,
        ],
    )
    def kern(s_hbm, src_hbm, dst_hbm, z1_hbm, t_hbm,
             idx_s, idx_d, vals, s_sp, t_sh, gsem, ssem):
        c = lax.axis_index("c")
        s = lax.axis_index("s")
        w = c * NS + s

        @pl.when(s == 0)
        def _():
            pltpu.sync_copy(z1_hbm, t_sh)

        @pl.when(s == 1)
        def _():
            pltpu.sync_copy(s_hbm, s_sp)

        pltpu.sync_copy(src_hbm.at[w], idx_s)
        pltpu.sync_copy(dst_hbm.at[w], idx_d)
        plsc.subcore_barrier()

        def start_gather(j, b):
            pltpu.async_copy(s_sp.at[idx_s.at[j]], vals[b], gsem[b])

        def wait_gather(j, b):
            pltpu.make_async_copy(s_sp.at[idx_s.at[j]], vals[b],
                                  gsem[b]).wait()

        def start_scatter(j, b):
            pltpu.async_copy(vals[b], t_sh.at[idx_d.at[j]], ssem[b],
                             add=True)

        def wait_scatter(j, b):
            pltpu.make_async_copy(vals[b], t_sh.at[idx_d.at[j]],
                                  ssem[b]).wait()

        for b in range(3):
            start_gather(b, b)

        def round_body(g, carry):
            for b in range(NBUF):
                j = g * NBUF + b
                wait_gather(j, b)
                start_scatter(j, b)
                if b >= 2:
                    wait_scatter(j - 2, b - 2)
                else:
                    @pl.when(g > 0)
                    def _():
                        wait_scatter(j - 2, (b - 2) % NBUF)
                if b < 2:
                    start_gather(j + 3, (b + 3) % NBUF)
                else:
                    @pl.when(g < GLOOP - 1)
                    def _():
                        start_gather(j + 3, (b + 3) % NBUF)
            return carry

        lax.fori_loop(0, GLOOP, round_body, 0)
        wait_scatter(RPT - 2, (RPT - 2) % NBUF)
        wait_scatter(RPT - 1, (RPT - 1) % NBUF)
        plsc.subcore_barrier()

        @pl.when(s == 0)
        def _():
            pltpu.sync_copy(t_sh, t_hbm.at[c])

    return kern(svec, src2d, dst2d, zeros_n)


BN = 1000


def _tc_dense1(agg2, deg, x, w1lt, b1l_r, w1rt, w2s):
    def body(agg_r, deg_r, x_r, w1l_r, b1l_r_, w1r_r,
             w2s_r, sh_r, degc_r):
        degc = jnp.maximum(deg_r[0] + deg_r[1], 1.0)
        mean = (agg_r[0] + agg_r[1]) / degc
        h = (jnp.dot(mean, w1l_r[...], preferred_element_type=jnp.float32)
             + b1l_r_[...]
             + jnp.dot(x_r[...], w1r_r[...], preferred_element_type=jnp.float32))
        h = jnp.maximum(h, 0.0)
        sh_r[...] = jnp.dot(h, w2s_r[...], preferred_element_type=jnp.float32)
        degc_r[...] = degc

    grid = (N // BN,)
    return pl.pallas_call(
        body,
        grid=grid,
        in_specs=[
            pl.BlockSpec((NC, BN, D), lambda i: (0, i, 0)),
            pl.BlockSpec((NC, BN, 1), lambda i: (0, i, 0)),
            pl.BlockSpec((BN, D), lambda i: (i, 0)),
            pl.BlockSpec((D, H), lambda i: (0, 0)),
            pl.BlockSpec((1, H), lambda i: (0, 0)),
            pl.BlockSpec((D, H), lambda i: (0, 0)),
            pl.BlockSpec((H, 2), lambda i: (0, 0)),
        ],
        out_specs=[
            pl.BlockSpec((BN, 2), lambda i: (i, 0)),
            pl.BlockSpec((BN, 1), lambda i: (i, 0)),
        ],
        out_shape=[
            jax.ShapeDtypeStruct((N, 2), jnp.float32),
            jax.ShapeDtypeStruct((N, 1), jnp.float32),
        ],
    )(agg2, deg, x, w1lt, b1l_r, w1rt, w2s)


def _tc_dense2(t0, t1, degc, hr, b2, wfc1, bfc1, wfc2, bfc2):
    def body(t0_r, t1_r, degc_r, hr_r, b2_r, wfc1_r, bfc1_r, wfc2_r, bfc2_r,
             p_r):
        v = jnp.maximum((t0_r[...] + t1_r[...]) / degc_r[...]
                        + b2_r[0, 0] + hr_r[...], 0.0)
        z = (jnp.dot(wfc1_r[...], v, preferred_element_type=jnp.float32)
             + bfc1_r[...])
        p_r[...] = (jnp.dot(wfc2_r[...], z,
                            preferred_element_type=jnp.float32)
                    + bfc2_r[...])

    return pl.pallas_call(
        body,
        out_shape=jax.ShapeDtypeStruct((1, 1), jnp.float32),
    )(t0, t1, degc, hr, b2, wfc1, bfc1, wfc2, bfc2)


def kernel(x, edge_index, W1l, b1l, W1r, W2l, b2l, W2r, Wfc1, bfc1, Wfc2, bfc2):
    src2d = edge_index[0].reshape(NW, RPT, GROUP)
    dst2d = edge_index[1].reshape(NW, RPT, GROUP)
    srcA = edge_index[0].reshape(NW, NCHUNK, CH, GA)
    dstA = edge_index[1].reshape(NW, NCHUNK, CH, GA)
    zeros_nd = jnp.zeros((N, D), jnp.float32)
    zeros_n = jnp.zeros((N,), jnp.float32)
    ones_g = jnp.ones((GA,), jnp.float32)

    agg2, deg = _sc_aggregate(x, srcA, dstA, zeros_nd, zeros_n, ones_g)

    sh, degc = _tc_dense1(
        agg2, deg.reshape(NC, N, 1),
        x, W1l.T, b1l.reshape(1, H), W1r.T,
        jnp.concatenate([W2l, W2r], axis=0).T,
    )

    s_vec = sh[:, 0]
    hr = sh[:, 1:2]

    t2 = _sc_edge_scalar(s_vec, src2d, dst2d, zeros_n)

    p = _tc_dense2(
        t2[0].reshape(N, 1), t2[1].reshape(N, 1), degc, hr,
        b2l.reshape(1, 1), Wfc1, bfc1.reshape(LH, 1), Wfc2,
        bfc2.reshape(1, 1),
    )
    return jnp.reshape(p, ())
